# baseline (speedup 1.0000x reference)
import functools
import jax
import jax.numpy as jnp
from jax import lax
from jax.experimental import pallas as pl
from jax.experimental.pallas import tpu as pltpu
from jax.experimental.pallas import tpu_sc as plsc

_N = 100000
_D = 512
_NW = 32
_B = 3200
_K = 64
_NCHUNK = _B // _K
_NV = 60
_BX = 4096


def _prep(x, w0, w1, w2):
    def body(x_ref, w0_ref, w1_ref, w2_ref, t_ref, idx_ref):
        @pl.when(pl.program_id(0) == 0)
        def _():
            for k in range(_NV):
                a, b, c = k // 12, (k // 2) % 6, k % 2
                t_ref[pl.ds(k, 1), :] = (
                    w0_ref[pl.ds(a, 1), :]
                    + w1_ref[pl.ds(b, 1), :]
                    + w2_ref[pl.ds(c, 1), :]
                )

        xb = x_ref[...]
        idx_ref[...] = xb[:, 0] * 12 + xb[:, 1] * 2 + xb[:, 2]

    return pl.pallas_call(
        body,
        grid=(pl.cdiv(_N, _BX),),
        in_specs=[
            pl.BlockSpec((_BX, 3), lambda i: (i, 0)),
            pl.BlockSpec((5, _D), lambda i: (0, 0)),
            pl.BlockSpec((6, _D), lambda i: (0, 0)),
            pl.BlockSpec((2, _D), lambda i: (0, 0)),
        ],
        out_specs=[
            pl.BlockSpec((_NV, _D), lambda i: (0, 0)),
            pl.BlockSpec((_BX,), lambda i: (i,)),
        ],
        out_shape=[
            jax.ShapeDtypeStruct((_NV, _D), jnp.float32),
            jax.ShapeDtypeStruct((_N,), jnp.int32),
        ],
    )(x, w0, w1, w2)


def _sc_lookup(tbl, idx):
    mesh = plsc.VectorSubcoreMesh(core_axis_name="c", subcore_axis_name="s")

    @functools.partial(
        pl.kernel,
        mesh=mesh,
        out_type=jax.ShapeDtypeStruct((_N, _D), jnp.float32),
        scratch_types=[
            pltpu.VMEM((_B,), jnp.int32),
            pltpu.VMEM((_NV, _D), jnp.float32),
            pltpu.VMEM((_K, _D), jnp.float32),
            pltpu.VMEM((_K, _D), jnp.float32),
            pltpu.SemaphoreType.DMA,
            pltpu.SemaphoreType.DMA,
        ],
    )
    def k(tbl_hbm, idx_hbm, out_hbm, idxv, tv, rows0, rows1, os0, os1):
        cid = lax.axis_index("c")
        sid = lax.axis_index("s")
        wid = sid * 2 + cid
        base = jnp.minimum(wid * _B, _N - _B)

        tcopy = pltpu.make_async_copy(tbl_hbm, tv, os0)
        tcopy.start()
        icopy = pltpu.make_async_copy(idx_hbm.at[pl.ds(base, _B)], idxv, os1)
        icopy.start()
        icopy.wait()
        tcopy.wait()

        def fill(c, buf):
            def grp(g, carry):
                sv = idxv[pl.ds(c * _K + g * 16, 16)]
                nj = _D // 16
                prev = [tv[sv[0], pl.ds(j * 16, 16)] for j in range(nj)]
                for l in range(1, 16):
                    s = sv[l]
                    cur = []
                    for j in range(nj):
                        v = tv[s, pl.ds(j * 16, 16)]
                        buf[g * 16 + l - 1, pl.ds(j * 16, 16)] = prev[j]
                        cur.append(v)
                    prev = cur
                for j in range(nj):
                    buf[g * 16 + 15, pl.ds(j * 16, 16)] = prev[j]
                return carry
            lax.fori_loop(0, _K // 16, grp, 0)

        def ocopy(c, buf, sem):
            return pltpu.make_async_copy(
                buf, out_hbm.at[pl.ds(base + c * _K, _K)], sem)

        fill(0, rows0)
        ocopy(0, rows0, os0).start()
        fill(1, rows1)
        ocopy(1, rows1, os1).start()

        def pair(g, carry):
            c0 = 2 + 2 * g
            ocopy(c0 - 2, rows0, os0).wait()
            fill(c0, rows0)
            ocopy(c0, rows0, os0).start()
            c1 = c0 + 1
            ocopy(c1 - 2, rows1, os1).wait()
            fill(c1, rows1)
            ocopy(c1, rows1, os1).start()
            return carry

        lax.fori_loop(0, (_NCHUNK - 2) // 2, pair, 0)
        ocopy(_NCHUNK - 2, rows0, os0).wait()
        ocopy(_NCHUNK - 1, rows1, os1).wait()

    return k(tbl, idx)


def kernel(x, W0, W1, W2):
    tbl, idx = _prep(x.astype(jnp.int32), W0, W1, W2)
    return _sc_lookup(tbl, idx)


# speedup vs baseline: 1.8662x; 1.8662x over previous
import functools
import jax
import jax.numpy as jnp
from jax import lax
from jax.experimental import pallas as pl
from jax.experimental.pallas import tpu as pltpu
from jax.experimental.pallas import tpu_sc as plsc

_N = 100000
_D = 512
_NW = 32
_B = 3200
_K = 64
_NCHUNK = _B // _K
_NV = 60


def _sc_lookup(w0, w1, w2, x0, x1, x2):
    mesh = plsc.VectorSubcoreMesh(core_axis_name="c", subcore_axis_name="s")

    @functools.partial(
        pl.kernel,
        mesh=mesh,
        out_type=jax.ShapeDtypeStruct((_N, _D), jnp.float32),
        scratch_types=[
            pltpu.VMEM((_B,), jnp.int32),
            pltpu.VMEM((_B,), jnp.int32),
            pltpu.VMEM((_B,), jnp.int32),
            pltpu.VMEM((_B,), jnp.int32),
            pltpu.VMEM((5, _D), jnp.float32),
            pltpu.VMEM((6, _D), jnp.float32),
            pltpu.VMEM((2, _D), jnp.float32),
            pltpu.VMEM((_NV, _D), jnp.float32),
            pltpu.VMEM((_K, _D), jnp.float32),
            pltpu.VMEM((_K, _D), jnp.float32),
            pltpu.SemaphoreType.DMA,
            pltpu.SemaphoreType.DMA,
        ],
    )
    def k(w0_hbm, w1_hbm, w2_hbm, x0_hbm, x1_hbm, x2_hbm, out_hbm,
          x0v, x1v, x2v, idxv, w0v, w1v, w2v, tv, rows0, rows1, os0, os1):
        cid = lax.axis_index("c")
        sid = lax.axis_index("s")
        wid = sid * 2 + cid
        base = jnp.minimum(wid * _B, _N - _B)

        t0 = pltpu.make_async_copy(w0_hbm, w0v, os0)
        t1 = pltpu.make_async_copy(w1_hbm, w1v, os0)
        t2 = pltpu.make_async_copy(w2_hbm, w2v, os0)
        t0.start(); t1.start(); t2.start()
        c0 = pltpu.make_async_copy(x0_hbm.at[pl.ds(base, _B)], x0v, os1)
        c1 = pltpu.make_async_copy(x1_hbm.at[pl.ds(base, _B)], x1v, os1)
        c2 = pltpu.make_async_copy(x2_hbm.at[pl.ds(base, _B)], x2v, os1)
        c0.start(); c1.start(); c2.start()
        t0.wait(); t1.wait(); t2.wait()

        def btab(j, carry):
            s = pl.ds(j * 16, 16)
            w0r = [w0v[a, s] for a in range(5)]
            w1r = [w1v[b, s] for b in range(6)]
            w2r = [w2v[c, s] for c in range(2)]
            t12 = [w1r[b] + w2r[c] for b in range(6) for c in range(2)]
            for a in range(5):
                for m in range(12):
                    tv[a * 12 + m, s] = w0r[a] + t12[m]
            return carry

        lax.fori_loop(0, _D // 16, btab, 0)
        c0.wait(); c1.wait(); c2.wait()

        def cidx(i, carry):
            s = pl.ds(i * 16, 16)
            idxv[s] = x0v[s] * 12 + x1v[s] * 2 + x2v[s]
            return carry

        lax.fori_loop(0, _B // 16, cidx, 0)

        def fill(c, buf):
            def grp(g, carry):
                sv = idxv[pl.ds(c * _K + g * 16, 16)]
                nj = _D // 16
                prev = [tv[sv[0], pl.ds(j * 16, 16)] for j in range(nj)]
                for l in range(1, 16):
                    s = sv[l]
                    cur = []
                    for j in range(nj):
                        v = tv[s, pl.ds(j * 16, 16)]
                        buf[g * 16 + l - 1, pl.ds(j * 16, 16)] = prev[j]
                        cur.append(v)
                    prev = cur
                for j in range(nj):
                    buf[g * 16 + 15, pl.ds(j * 16, 16)] = prev[j]
                return carry
            lax.fori_loop(0, _K // 16, grp, 0)

        def ocopy(c, buf, sem):
            return pltpu.make_async_copy(
                buf, out_hbm.at[pl.ds(base + c * _K, _K)], sem)

        fill(0, rows0)
        ocopy(0, rows0, os0).start()
        fill(1, rows1)
        ocopy(1, rows1, os1).start()

        def pair(g, carry):
            c0 = 2 + 2 * g
            ocopy(c0 - 2, rows0, os0).wait()
            fill(c0, rows0)
            ocopy(c0, rows0, os0).start()
            c1 = c0 + 1
            ocopy(c1 - 2, rows1, os1).wait()
            fill(c1, rows1)
            ocopy(c1, rows1, os1).start()
            return carry

        lax.fori_loop(0, (_NCHUNK - 2) // 2, pair, 0)
        ocopy(_NCHUNK - 2, rows0, os0).wait()
        ocopy(_NCHUNK - 1, rows1, os1).wait()

    return k(w0, w1, w2, x0, x1, x2)


def kernel(x, W0, W1, W2):
    xt = x.T.astype(jnp.int32)
    return _sc_lookup(W0, W1, W2, xt[0], xt[1], xt[2])
